# baseline (speedup 1.0000x reference)
import jax
import jax.numpy as jnp
from jax.experimental import pallas as pl
from jax.experimental.pallas import tpu as pltpu

_BH = 64
_NS = 2


def _acke_body(x_ref, *refs):
    nw = refs[:_NS]
    ow = refs[_NS:2 * _NS]
    o1_ref, o2_ref = refs[2 * _NS], refs[2 * _NS + 1]
    x = x_ref[...]
    dims = (((1,), (1,)), ((), ()))
    for i in range(_NS):
        o1_ref[:, i * _BH:(i + 1) * _BH] = jax.lax.dot_general(
            x, nw[i][...], dims, preferred_element_type=jnp.float32)
        o2_ref[:, i * _BH:(i + 1) * _BH] = jax.lax.dot_general(
            x, ow[i][...], dims, preferred_element_type=jnp.float32)


def _wspec(i):
    return pl.BlockSpec((_BH, 4096), lambda j, i=i: (_NS * j + i, 0))


@jax.jit
def kernel(x, new_weight, orig_weight):
    b, k = x.shape
    n = new_weight.shape[0]
    bn = _BH * _NS
    grid = (n // bn,)
    out_shape = jax.ShapeDtypeStruct((b, n), jnp.float32)
    call = pl.pallas_call(
        _acke_body,
        grid=grid,
        in_specs=[pl.BlockSpec((b, k), lambda j: (0, 0))]
        + [_wspec(i) for i in range(_NS)] * 2,
        out_specs=[
            pl.BlockSpec((b, bn), lambda j: (0, j)),
            pl.BlockSpec((b, bn), lambda j: (0, j)),
        ],
        out_shape=[out_shape, out_shape],
        compiler_params=pltpu.CompilerParams(
            dimension_semantics=("arbitrary",)),
    )
    layer_out, original_layer_output = call(
        x, *([new_weight] * _NS), *([orig_weight] * _NS))
    return (layer_out, original_layer_output)


# speedup vs baseline: 1.2006x; 1.2006x over previous
import jax
import jax.numpy as jnp
from jax.experimental import pallas as pl
from jax.experimental.pallas import tpu as pltpu

_BH = 256
_NS = 2


def _acke_body(x_ref, *refs):
    nw = refs[:_NS]
    ow = refs[_NS:2 * _NS]
    o1_ref, o2_ref = refs[2 * _NS], refs[2 * _NS + 1]
    x = x_ref[...]
    dims = (((1,), (1,)), ((), ()))
    for i in range(_NS):
        o1_ref[:, i * _BH:(i + 1) * _BH] = jax.lax.dot_general(
            x, nw[i][...], dims, preferred_element_type=jnp.float32)
        o2_ref[:, i * _BH:(i + 1) * _BH] = jax.lax.dot_general(
            x, ow[i][...], dims, preferred_element_type=jnp.float32)


def _wspec(i):
    return pl.BlockSpec((_BH, 4096), lambda j, i=i: (_NS * j + i, 0))


@jax.jit
def kernel(x, new_weight, orig_weight):
    b, k = x.shape
    n = new_weight.shape[0]
    bn = _BH * _NS
    grid = (n // bn,)
    out_shape = jax.ShapeDtypeStruct((b, n), jnp.float32)
    call = pl.pallas_call(
        _acke_body,
        grid=grid,
        in_specs=[pl.BlockSpec((b, k), lambda j: (0, 0))]
        + [_wspec(i) for i in range(_NS)] * 2,
        out_specs=[
            pl.BlockSpec((b, bn), lambda j: (0, j)),
            pl.BlockSpec((b, bn), lambda j: (0, j)),
        ],
        out_shape=[out_shape, out_shape],
        compiler_params=pltpu.CompilerParams(
            dimension_semantics=("arbitrary",)),
    )
    layer_out, original_layer_output = call(
        x, *([new_weight] * _NS), *([orig_weight] * _NS))
    return (layer_out, original_layer_output)
